# baseline (speedup 1.0000x reference)
import functools

import jax
import jax.numpy as jnp
from jax import lax
from jax.experimental import pallas as pl
from jax.experimental.pallas import tpu as pltpu
from jax.experimental.pallas import tpu_sc as plsc

_N = 50000
_K = 7
_F = 128
_NW = 32
_C = 16
_G = _C * _K
_RPW = 1568
_NCHUNK = _RPW // _C
_NBUF = 2
_BN = 1000


def _tc_matmul(x, wcat, bias2d):
    def body(x_ref, w_ref, b_ref, y_ref):
        xb = x_ref[...].astype(jnp.bfloat16)
        y_ref[...] = jnp.dot(xb, w_ref[...],
                             preferred_element_type=jnp.float32) + b_ref[...]

    return pl.pallas_call(
        body,
        grid=(_N // _BN,),
        in_specs=[
            pl.BlockSpec((_BN, _F), lambda i: (i, 0)),
            pl.BlockSpec((_F, _K * _F), lambda i: (0, 0)),
            pl.BlockSpec((1, _K * _F), lambda i: (0, 0)),
        ],
        out_specs=pl.BlockSpec((_BN, _K * _F), lambda i: (i, 0)),
        out_shape=jax.ShapeDtypeStruct((_N, _K * _F), jnp.float32),
    )(x, wcat, bias2d)


def _sc_gather_accum(y_flat, idx):
    mesh = plsc.VectorSubcoreMesh(core_axis_name="c", subcore_axis_name="s")

    @functools.partial(
        pl.kernel,
        mesh=mesh,
        out_type=jax.ShapeDtypeStruct((_N, _F), jnp.float32),
        scratch_types=[
            pltpu.VMEM((_RPW * _K,), jnp.int32),
            pltpu.VMEM((_RPW * _K,), jnp.int32),
            pltpu.VMEM((_NBUF, _G, _F), jnp.float32),
            pltpu.VMEM((_NBUF, _C, _F), jnp.float32),
            pltpu.SemaphoreType.DMA,
            pltpu.SemaphoreType.DMA,
            pltpu.SemaphoreType.DMA,
            pltpu.SemaphoreType.DMA,
        ],
    )
    def k(y_hbm, idx_hbm, out_hbm, raw_v, adj_v, rows_v, out_v,
          gsem0, gsem1, osem0, osem1):
        gsems = (gsem0, gsem1)
        osems = (osem0, osem1)
        wid = lax.axis_index("s") * 2 + lax.axis_index("c")
        wbase = jnp.minimum(wid * _RPW, _N - _RPW)

        pltpu.sync_copy(idx_hbm.at[pl.ds(wbase * _K, _RPW * _K)], raw_v)

        def adj_body(g, carry):
            l = g * 16 + lax.iota(jnp.int32, 16)
            adj_v[pl.ds(g * 16, 16)] = (
                raw_v[pl.ds(g * 16, 16)] * _K + lax.rem(l, _K))
            return carry

        lax.fori_loop(0, _RPW * _K // 16, adj_body, 0)

        def start_gather(c, b):
            return pltpu.async_copy(
                y_hbm.at[adj_v.at[pl.ds(c * _G, _G)]], rows_v.at[b], gsems[b])

        for b in range(_NBUF):
            start_gather(b, b)

        def pair_body(p, carry):
            for b in range(_NBUF):
                c = p * _NBUF + b
                pltpu.make_async_copy(
                    y_hbm.at[adj_v.at[pl.ds(0, _G)]], rows_v.at[b],
                    gsems[b]).wait()
                @pl.when(p > 0)
                def _():
                    pltpu.make_async_copy(
                        out_v.at[b], out_hbm.at[pl.ds(0, _C)],
                        osems[b]).wait()

                def row_body(r, rcarry):
                    rb = r * _K
                    for v in range(_F // 16):
                        sl = pl.ds(v * 16, 16)
                        acc = rows_v[b, rb, sl]
                        for kk in range(1, _K):
                            acc = acc + rows_v[b, rb + kk, sl]
                        out_v[b, r, sl] = acc
                    return rcarry

                lax.fori_loop(0, _C, row_body, 0)

                @pl.when(c + _NBUF < _NCHUNK)
                def _():
                    start_gather(c + _NBUF, b)

                pltpu.async_copy(
                    out_v.at[b], out_hbm.at[pl.ds(wbase + c * _C, _C)],
                    osems[b])
            return carry

        lax.fori_loop(0, _NCHUNK // _NBUF, pair_body, 0)

        for b in range(_NBUF):
            pltpu.make_async_copy(
                out_v.at[b], out_hbm.at[pl.ds(0, _C)], osems[b]).wait()

    return k(y_flat, idx)


def kernel(x, neigh_orders, W, b):
    wcat = W.reshape(_F, _K, _F).transpose(2, 1, 0).reshape(_F, _K * _F)
    bias = jnp.tile(b / float(_K), _K).reshape(1, _K * _F)
    y = _tc_matmul(x, wcat.astype(jnp.bfloat16), bias)
    y_flat = y.reshape(_N * _K, _F)
    return _sc_gather_accum(y_flat, neigh_orders.astype(jnp.int32))


# speedup vs baseline: 1.0377x; 1.0377x over previous
import functools

import jax
import jax.numpy as jnp
from jax import lax
from jax.experimental import pallas as pl
from jax.experimental.pallas import tpu as pltpu
from jax.experimental.pallas import tpu_sc as plsc

_N = 50000
_K = 7
_F = 128
_NW = 32
_C = 16
_G = _C * _K
_RPW = 1568
_NCHUNK = _RPW // _C
_NBUF = 2
_BN = 5000


def _tc_matmul(x, wcat, bias2d):
    def body(x_ref, w_ref, b_ref, y_ref):
        xb = x_ref[...].astype(jnp.bfloat16)
        y_ref[...] = jnp.dot(xb, w_ref[...],
                             preferred_element_type=jnp.float32) + b_ref[...]

    return pl.pallas_call(
        body,
        grid=(_N // _BN,),
        in_specs=[
            pl.BlockSpec((_BN, _F), lambda i: (i, 0)),
            pl.BlockSpec((_F, _K * _F), lambda i: (0, 0)),
            pl.BlockSpec((1, _K * _F), lambda i: (0, 0)),
        ],
        out_specs=pl.BlockSpec((_BN, _K * _F), lambda i: (i, 0)),
        out_shape=jax.ShapeDtypeStruct((_N, _K * _F), jnp.float32),
    )(x, wcat, bias2d)


def _sc_gather_accum(y_flat, idx):
    mesh = plsc.VectorSubcoreMesh(core_axis_name="c", subcore_axis_name="s")

    @functools.partial(
        pl.kernel,
        mesh=mesh,
        out_type=jax.ShapeDtypeStruct((_N, _F), jnp.float32),
        scratch_types=[
            pltpu.VMEM((_RPW * _K,), jnp.int32),
            pltpu.VMEM((_RPW * _K,), jnp.int32),
            pltpu.VMEM((_NBUF, _G, _F), jnp.float32),
            pltpu.VMEM((_NBUF, _C, _F), jnp.float32),
            pltpu.SemaphoreType.DMA,
            pltpu.SemaphoreType.DMA,
            pltpu.SemaphoreType.DMA,
            pltpu.SemaphoreType.DMA,
        ],
    )
    def k(y_hbm, idx_hbm, out_hbm, raw_v, adj_v, rows_v, out_v,
          gsem0, gsem1, osem0, osem1):
        gsems = (gsem0, gsem1)
        osems = (osem0, osem1)
        wid = lax.axis_index("s") * 2 + lax.axis_index("c")
        wbase = jnp.minimum(wid * _RPW, _N - _RPW)

        pltpu.sync_copy(idx_hbm.at[pl.ds(wbase * _K, _RPW * _K)], raw_v)

        def adj_body(g, carry):
            l = g * 16 + lax.iota(jnp.int32, 16)
            adj_v[pl.ds(g * 16, 16)] = (
                raw_v[pl.ds(g * 16, 16)] * _K + lax.rem(l, _K))
            return carry

        lax.fori_loop(0, _RPW * _K // 16, adj_body, 0)

        def start_gather(c, b):
            return pltpu.async_copy(
                y_hbm.at[adj_v.at[pl.ds(c * _G, _G)]], rows_v.at[b], gsems[b])

        for b in range(_NBUF):
            start_gather(b, b)

        def pair_body(p, carry):
            for b in range(_NBUF):
                c = p * _NBUF + b
                pltpu.make_async_copy(
                    y_hbm.at[adj_v.at[pl.ds(0, _G)]], rows_v.at[b],
                    gsems[b]).wait()
                @pl.when(p > 0)
                def _():
                    pltpu.make_async_copy(
                        out_v.at[b], out_hbm.at[pl.ds(0, _C)],
                        osems[b]).wait()

                def row_body(r, rcarry):
                    rb = r * _K
                    for v in range(_F // 16):
                        sl = pl.ds(v * 16, 16)
                        acc = rows_v[b, rb, sl]
                        for kk in range(1, _K):
                            acc = acc + rows_v[b, rb + kk, sl]
                        out_v[b, r, sl] = acc
                    return rcarry

                lax.fori_loop(0, _C, row_body, 0)

                @pl.when(c + _NBUF < _NCHUNK)
                def _():
                    start_gather(c + _NBUF, b)

                pltpu.async_copy(
                    out_v.at[b], out_hbm.at[pl.ds(wbase + c * _C, _C)],
                    osems[b])
            return carry

        lax.fori_loop(0, _NCHUNK // _NBUF, pair_body, 0)

        for b in range(_NBUF):
            pltpu.make_async_copy(
                out_v.at[b], out_hbm.at[pl.ds(0, _C)], osems[b]).wait()

    return k(y_flat, idx)


def kernel(x, neigh_orders, W, b):
    wcat = W.reshape(_F, _K, _F).transpose(2, 1, 0).reshape(_F, _K * _F)
    bias = jnp.tile(b / float(_K), _K).reshape(1, _K * _F)
    y = _tc_matmul(x, wcat.astype(jnp.bfloat16), bias)
    y_flat = y.reshape(_N * _K, _F)
    return _sc_gather_accum(y_flat, neigh_orders.astype(jnp.int32))
